# Initial kernel scaffold; baseline (speedup 1.0000x reference)
import jax
import jax.numpy as jnp
from jax.experimental import pallas as pl


def kernel(x, edge_index, edge_attr, node_feature, W1, b1, W2, b2, W3, b3, W4, b4):
    raise NotImplementedError("write your pallas kernel here")


# speedup vs baseline: 9.9393x; 9.9393x over previous
import functools

import jax
import jax.numpy as jnp
from jax import lax
from jax.experimental import pallas as pl
from jax.experimental.pallas import tpu as pltpu
from jax.experimental.pallas import tpu_sc as plsc

_N = 10000
_NP = 10240
_E = 320000
_DIN = 128
_NC = 2
_NS = 16
_NW = _NC * _NS
_C = 128
_EPT = 10240
_NCHUNK = _EPT // _C
_EPAD = _NW * _EPT
_RPS = _NP // _NS

_MESH = dict(core_axis_name="c", subcore_axis_name="s")


def _zero_rows(buf, nrows, ncols):
    def body(i, _):
        for j in range(ncols // 16):
            buf[i, pl.ds(j * 16, 16)] = jnp.zeros((16,), jnp.float32)
        return 0
    lax.fori_loop(0, nrows, body, 0)


def _make_deg():
    @functools.partial(
        pl.kernel,
        out_type=jax.ShapeDtypeStruct((_NC, _NP, 16), jnp.float32),
        mesh=plsc.VectorSubcoreMesh(**_MESH),
        scratch_types=[
            pltpu.VMEM((_NCHUNK, _C), jnp.int32),
            pltpu.VMEM((_C, 16), jnp.float32),
            pltpu.VMEM_SHARED((_NP, 16), jnp.float32),
        ],
    )
    def deg(dst_hbm, out_hbm, dst_v, obuf, acc):
        cid = lax.axis_index("c")
        sid = lax.axis_index("s")
        wid = sid * _NC + cid
        _zero_rows(obuf, _C, 16)
        for r in range(_RPS // _C):
            pltpu.sync_copy(obuf, acc.at[pl.ds(sid * _RPS + r * _C, _C)])
        def ones(i, _):
            obuf[i] = jnp.ones((16,), jnp.float32)
            return 0
        lax.fori_loop(0, _C, ones, 0)
        pltpu.sync_copy(dst_hbm.at[wid], dst_v)
        plsc.subcore_barrier()
        def body(j, _):
            pltpu.sync_copy(obuf, acc.at[dst_v.at[j]], add=True)
            return 0
        lax.fori_loop(0, _NCHUNK, body, 0)
        plsc.subcore_barrier()
        pltpu.sync_copy(acc.at[pl.ds(sid * _RPS, _RPS)],
                        out_hbm.at[cid, pl.ds(sid * _RPS, _RPS)])

    return deg


def _make_spmm(F):
    @functools.partial(
        pl.kernel,
        out_type=jax.ShapeDtypeStruct((_NC, _NP, F), jnp.float32),
        mesh=plsc.VectorSubcoreMesh(**_MESH),
        scratch_types=[
            pltpu.VMEM((_NCHUNK, _C), jnp.int32),
            pltpu.VMEM((_NCHUNK, _C), jnp.int32),
            pltpu.VMEM((_C, F), jnp.float32),
            pltpu.VMEM_SHARED((_NP, F), jnp.float32),
            pltpu.SemaphoreType.DMA,
        ],
        compiler_params=pltpu.CompilerParams(use_tc_tiling_on_sc=False),
    )
    def spmm(y_hbm, src_hbm, dst_hbm, out_hbm, src_v, dst_v, gbuf, acc, sem):
        cid = lax.axis_index("c")
        sid = lax.axis_index("s")
        wid = sid * _NC + cid
        _zero_rows(gbuf, _C, F)
        for r in range(_RPS // _C):
            pltpu.sync_copy(gbuf, acc.at[pl.ds(sid * _RPS + r * _C, _C)])
        pltpu.sync_copy(src_hbm.at[wid], src_v)
        pltpu.sync_copy(dst_hbm.at[wid], dst_v)
        plsc.subcore_barrier()
        def body(j, _):
            pltpu.async_copy(y_hbm.at[src_v.at[j]], gbuf, sem).wait()
            pltpu.sync_copy(gbuf, acc.at[dst_v.at[j]], add=True)
            return 0
        lax.fori_loop(0, _NCHUNK, body, 0)
        plsc.subcore_barrier()
        pltpu.sync_copy(acc.at[pl.ds(sid * _RPS, _RPS)],
                        out_hbm.at[cid, pl.ds(sid * _RPS, _RPS)])

    return spmm


_deg_kernel = _make_deg()
_spmm_16 = _make_spmm(16)
_spmm_64 = _make_spmm(64)
_spmm_128 = _make_spmm(128)

_R = 1024


def _row_valid(g):
    rows = lax.broadcasted_iota(jnp.int32, (_R, 1), 0) + g * _R
    return rows < _N


def _prep_body(deg_ref, x_ref, nf_ref, t_ref, dinv_ref):
    g = pl.program_id(0)
    deg = deg_ref[0, :, :1] + deg_ref[1, :, :1] + 1.0
    valid = _row_valid(g).astype(jnp.float32)
    dinv = lax.rsqrt(jnp.maximum(deg, 1e-12)) * valid
    x = x_ref[...]
    e0 = jnp.all(x == nf_ref[0:1, :], axis=1, keepdims=True)
    e1 = jnp.all(x == nf_ref[1:2, :], axis=1, keepdims=True)
    e2 = jnp.all(x == nf_ref[2:3, :], axis=1, keepdims=True)
    c0 = e0
    c1 = e1 & ~e0
    c2 = e2 & ~(e0 | e1)
    c3 = ~(e0 | e1 | e2)
    col = lax.broadcasted_iota(jnp.int32, (_R, 16), 1)
    t = jnp.zeros((_R, 16), jnp.float32)
    for k, c in enumerate((c0, c1, c2, c3)):
        t = t + jnp.where(col == k, c.astype(jnp.float32), 0.0)
    t_ref[...] = t * dinv
    dinv_ref[...] = jnp.broadcast_to(dinv, (_R, 16))


def _prep(degp, xp, nf):
    grid = _NP // _R
    return pl.pallas_call(
        _prep_body,
        grid=(grid,),
        in_specs=[
            pl.BlockSpec((_NC, _R, 16), lambda g: (0, g, 0)),
            pl.BlockSpec((_R, _DIN), lambda g: (g, 0)),
            pl.BlockSpec((4, _DIN), lambda g: (0, 0)),
        ],
        out_specs=[
            pl.BlockSpec((_R, 16), lambda g: (g, 0)),
            pl.BlockSpec((_R, 16), lambda g: (g, 0)),
        ],
        out_shape=[
            jax.ShapeDtypeStruct((_NP, 16), jnp.float32),
            jax.ShapeDtypeStruct((_NP, 16), jnp.float32),
        ],
    )(degp, xp, nf)


def _layer1_body(coef_ref, t_ref, dinv_ref, nf_ref, w1_ref, b1_ref, w2_ref,
                 y2_ref):
    dv = dinv_ref[:, :1]
    xw1 = jnp.dot(nf_ref[...], w1_ref[...],
                  preferred_element_type=jnp.float32)
    c = (coef_ref[0, :, :4] + coef_ref[1, :, :4] + t_ref[:, :4]) * dv
    h = jnp.maximum(jnp.dot(c, xw1, preferred_element_type=jnp.float32)
                    + b1_ref[...], 0.0)
    y2_ref[...] = dv * jnp.dot(h, w2_ref[...],
                               preferred_element_type=jnp.float32)


def _layer1(coefp, t, dinv, nf, W1, b1, W2):
    grid = _NP // _R
    return pl.pallas_call(
        _layer1_body,
        grid=(grid,),
        in_specs=[
            pl.BlockSpec((_NC, _R, 16), lambda g: (0, g, 0)),
            pl.BlockSpec((_R, 16), lambda g: (g, 0)),
            pl.BlockSpec((_R, 16), lambda g: (g, 0)),
            pl.BlockSpec((4, _DIN), lambda g: (0, 0)),
            pl.BlockSpec(W1.shape, lambda g: (0, 0)),
            pl.BlockSpec((1, b1.shape[1]), lambda g: (0, 0)),
            pl.BlockSpec(W2.shape, lambda g: (0, 0)),
        ],
        out_specs=pl.BlockSpec((_R, W2.shape[1]), lambda g: (g, 0)),
        out_shape=jax.ShapeDtypeStruct((_NP, W2.shape[1]), jnp.float32),
    )(coefp, t, dinv, nf, W1, b1, W2)


def _layer_body(z_ref, y_ref, dinv_ref, b_ref, w_ref, o_ref):
    dv = dinv_ref[:, :1]
    h = jnp.maximum(dv * (z_ref[0] + z_ref[1] + y_ref[...]) + b_ref[...],
                    0.0)
    o_ref[...] = dv * jnp.dot(h, w_ref[...],
                              preferred_element_type=jnp.float32)


def _layer(zp, y, dinv, b, W):
    grid = _NP // _R
    din, dout = W.shape
    return pl.pallas_call(
        _layer_body,
        grid=(grid,),
        in_specs=[
            pl.BlockSpec((_NC, _R, din), lambda g: (0, g, 0)),
            pl.BlockSpec((_R, din), lambda g: (g, 0)),
            pl.BlockSpec((_R, 16), lambda g: (g, 0)),
            pl.BlockSpec((1, din), lambda g: (0, 0)),
            pl.BlockSpec((din, dout), lambda g: (0, 0)),
        ],
        out_specs=pl.BlockSpec((_R, dout), lambda g: (g, 0)),
        out_shape=jax.ShapeDtypeStruct((_NP, dout), jnp.float32),
    )(zp, y, dinv, b, W)


def _pool_body(z_ref, y_ref, dinv_ref, b_ref, x_ref, nf_ref, o_ref):
    g = pl.program_id(0)

    @pl.when(g == 0)
    def _init():
        o_ref[...] = jnp.zeros_like(o_ref)

    dv = dinv_ref[:, :1]
    h = jnp.maximum(dv * (z_ref[0] + z_ref[1] + y_ref[...]) + b_ref[...],
                    0.0)
    valid = _row_valid(g)
    x = x_ref[...]
    e0 = jnp.all(x == nf_ref[0:1, :], axis=1, keepdims=True)
    e1 = jnp.all(x == nf_ref[1:2, :], axis=1, keepdims=True)
    e2 = jnp.all(x == nf_ref[2:3, :], axis=1, keepdims=True)
    m0 = e0 & valid
    m1 = e1 & valid
    m2 = e2 & valid
    m3 = valid & ~(e0 | e1 | e2)
    for k, m in enumerate((m0, m2, m3, m1)):
        mf = m.astype(jnp.float32)
        o_ref[k:k + 1, :] += jnp.sum(h * mf, axis=0, keepdims=True)
        o_ref[k + 4:k + 5, :] += jnp.sum(mf)

    @pl.when(g == pl.num_programs(0) - 1)
    def _final():
        o_ref[0:4, :] = o_ref[0:4, :] / jnp.maximum(o_ref[4:8, :], 1.0)


def _pool(zp, y, dinv, b, xp, nf):
    grid = _NP // _R
    return pl.pallas_call(
        _pool_body,
        grid=(grid,),
        in_specs=[
            pl.BlockSpec((_NC, _R, 128), lambda g: (0, g, 0)),
            pl.BlockSpec((_R, 128), lambda g: (g, 0)),
            pl.BlockSpec((_R, 16), lambda g: (g, 0)),
            pl.BlockSpec((1, 128), lambda g: (0, 0)),
            pl.BlockSpec((_R, _DIN), lambda g: (g, 0)),
            pl.BlockSpec((4, _DIN), lambda g: (0, 0)),
        ],
        out_specs=pl.BlockSpec((8, 128), lambda g: (0, 0)),
        out_shape=jax.ShapeDtypeStruct((8, 128), jnp.float32),
    )(zp, y, dinv, b, xp, nf)


def kernel(x, edge_index, edge_attr, node_feature, W1, b1, W2, b2, W3, b3,
           W4, b4):
    src = edge_index[0].astype(jnp.int32)
    dst = edge_index[1].astype(jnp.int32)
    pad = jnp.full((_EPAD - _E,), _N, jnp.int32)
    srcp = jnp.concatenate([src, pad]).reshape(_NW, _NCHUNK, _C)
    dstp = jnp.concatenate([dst, pad]).reshape(_NW, _NCHUNK, _C)
    xp = jnp.zeros((_NP, _DIN), jnp.float32).at[:_N].set(x)

    degp = _deg_kernel(dstp)
    t, dinv = _prep(degp, xp, node_feature)
    coefp = _spmm_16(t, srcp, dstp)
    y2 = _layer1(coefp, t, dinv, node_feature, W1,
                 b1.reshape(1, -1), W2)
    z2 = _spmm_128(y2, srcp, dstp)
    y3 = _layer(z2, y2, dinv, b2.reshape(1, -1), W3)
    z3 = _spmm_64(y3, srcp, dstp)
    y4 = _layer(z3, y3, dinv, b3.reshape(1, -1), W4)
    z4 = _spmm_128(y4, srcp, dstp)
    pooled = _pool(z4, y4, dinv, b4.reshape(1, -1), xp, node_feature)
    return pooled[:4].reshape(1, 512)
